# Initial kernel scaffold; baseline (speedup 1.0000x reference)
import jax
import jax.numpy as jnp
from jax.experimental import pallas as pl


def kernel(x, edge_index, W1, b1, W2, b2, W3, b3):
    raise NotImplementedError("write your pallas kernel here")


# speedup vs baseline: 6.5732x; 6.5732x over previous
import functools

import jax
import jax.numpy as jnp
from jax import lax
from jax.experimental import pallas as pl
from jax.experimental.pallas import tpu as pltpu
from jax.experimental.pallas import tpu_sc as plsc

N = 10000
NP = 10240
SUB_ROWS = NP // 16
E = 320000
CH = 128
E_PAD = 327680
C_A = E_PAD // (32 * CH)
C_B = (2 * E_PAD) // (32 * CH)
TRASH = NP - 1
BM = 256
GRID = NP // BM

_mesh = functools.partial(
    plsc.VectorSubcoreMesh, core_axis_name="c", subcore_axis_name="s")


IB = 16


def _make_prop(C):
  assert C % IB == 0

  @functools.partial(
      pl.kernel,
      out_type=jax.ShapeDtypeStruct((32, SUB_ROWS, 128), jnp.float32),
      mesh=_mesh(),
      scratch_types=[
          pltpu.VMEM_SHARED((NP, 128), jnp.float32),
          pltpu.VMEM((IB, CH), jnp.int32),
          pltpu.VMEM((IB, CH), jnp.int32),
          pltpu.VMEM((CH, 128), jnp.float32),
          pltpu.VMEM((CH, 128), jnp.float32),
          pltpu.SemaphoreType.DMA,
          pltpu.SemaphoreType.DMA,
      ],
  )
  def prop(table, src_idx, dst_idx, zeros, out,
           acc, src_v, dst_v, buf0, buf1, sem0, sem1):
    c = lax.axis_index("c")
    s = lax.axis_index("s")
    wid = c * 16 + s
    my_src = src_idx.at[wid]
    my_dst = dst_idx.at[wid]
    pltpu.sync_copy(zeros, acc.at[pl.ds(s * SUB_ROWS, SUB_ROWS)])
    plsc.subcore_barrier()

    @pl.loop(0, C, step=IB)
    def _(j0):
      pltpu.sync_copy(my_src.at[pl.ds(j0, IB)], src_v)
      pltpu.sync_copy(my_dst.at[pl.ds(j0, IB)], dst_v)

      @pl.loop(0, IB, step=2)
      def _(t):
        cp0 = pltpu.async_copy(table.at[src_v.at[t]], buf0, sem0)
        cp1 = pltpu.async_copy(table.at[src_v.at[t + 1]], buf1, sem1)
        cp0.wait()
        pltpu.sync_copy(buf0, acc.at[dst_v.at[t]], add=True)
        cp1.wait()
        pltpu.sync_copy(buf1, acc.at[dst_v.at[t + 1]], add=True)

    plsc.subcore_barrier()
    pltpu.sync_copy(acc.at[pl.ds(s * SUB_ROWS, SUB_ROWS)], out.at[wid])

  return prop


_prop_half_edges = _make_prop(C_A)
_prop_half_cols = _make_prop(C_B)


@functools.partial(
    pl.kernel,
    out_type=jax.ShapeDtypeStruct((32, SUB_ROWS, 16), jnp.float32),
    mesh=_mesh(),
    scratch_types=[
        pltpu.VMEM_SHARED((NP, 16), jnp.float32),
        pltpu.VMEM((C_A, CH), jnp.int32),
        pltpu.VMEM((CH, 16), jnp.float32),
    ],
)
def _deg_kernel(dst_idx, zeros16, ones16, out, acc, dst_v, ones_v):
  c = lax.axis_index("c")
  s = lax.axis_index("s")
  wid = c * 16 + s
  pltpu.sync_copy(dst_idx.at[wid], dst_v)
  pltpu.sync_copy(ones16, ones_v)
  pltpu.sync_copy(zeros16, acc.at[pl.ds(s * SUB_ROWS, SUB_ROWS)])
  plsc.subcore_barrier()

  @pl.loop(0, C_A)
  def _(j):
    pltpu.sync_copy(ones_v, acc.at[dst_v.at[j]], add=True)

  plsc.subcore_barrier()
  pltpu.sync_copy(acc.at[pl.ds(s * SUB_ROWS, SUB_ROWS)], out.at[wid])


def _dis_of(deg):
  return lax.rsqrt(deg[0, :, :1] + deg[1, :, :1] + 1.0)


def _k1_body(deg_ref, x_ref, o_ref):
  o_ref[...] = x_ref[...] * _dis_of(deg_ref[...])


def _k2_body(deg_ref, acc_ref, xt_ref, w1_ref, b1_ref, o_ref):
  dis = _dis_of(deg_ref[...])
  a = acc_ref[...]
  z = (a[0] + a[1] + xt_ref[...]) * dis
  h = jnp.dot(z, w1_ref[...], preferred_element_type=jnp.float32)
  h = jnp.maximum(h + b1_ref[...], 0.0)
  ht = h * dis
  o_ref[...] = jnp.stack([ht[:, :128], ht[:, 128:]])


def _k3_body(deg_ref, acc_ref, ht_ref, w2_ref, b2_ref, w3_ref, o_ref):
  dis = _dis_of(deg_ref[...])
  a = acc_ref[...]
  ht = ht_ref[...]
  z = jnp.concatenate([(a[0] + ht[0]) * dis, (a[1] + ht[1]) * dis], axis=1)
  h = jnp.dot(z, w2_ref[...], preferred_element_type=jnp.float32)
  h = jnp.maximum(h + b2_ref[...], 0.0)
  g = jnp.dot(h, w3_ref[...], preferred_element_type=jnp.float32)
  o_ref[...] = g * dis


def _k4_body(deg_ref, acc_ref, gt_ref, b3_ref, o_ref):
  dis = _dis_of(deg_ref[...])
  a = acc_ref[...]
  y = (a[0] + a[1] + gt_ref[...]) * dis + b3_ref[...]
  o_ref[...] = jnp.maximum(y, 0.0)


_DEG_SPEC = pl.BlockSpec((2, BM, 16), lambda i: (0, i, 0))
_ROW_SPEC = pl.BlockSpec((BM, 128), lambda i: (i, 0))
_ACC_SPEC = pl.BlockSpec((2, BM, 128), lambda i: (0, i, 0))


def _full(shape):
  return pl.BlockSpec(shape, lambda i: tuple(0 for _ in shape))


def kernel(x, edge_index, W1, b1, W2, b2, W3, b3):
  f32 = jnp.float32
  xp = jnp.pad(x, ((0, NP - N), (0, 0)))
  src = edge_index[0]
  dst = edge_index[1]
  padv = jnp.full((E_PAD - E,), TRASH, dtype=src.dtype)
  srcp = jnp.concatenate([src, padv])
  dstp = jnp.concatenate([dst, padv])
  srcA = srcp.reshape(32, C_A, CH)
  dstA = dstp.reshape(32, C_A, CH)
  srcB = jnp.concatenate([srcp, srcp + NP]).reshape(32, C_B, CH)
  dstB = jnp.concatenate([dstp, dstp]).reshape(32, C_B, CH)
  z128 = jnp.zeros((SUB_ROWS, 128), f32)
  z16 = jnp.zeros((SUB_ROWS, 16), f32)
  o16 = jnp.ones((CH, 16), f32)

  deg = _deg_kernel(dstA, z16, o16).reshape(2, NP, 16)

  xt = pl.pallas_call(
      _k1_body, grid=(GRID,),
      in_specs=[_DEG_SPEC, _ROW_SPEC],
      out_specs=_ROW_SPEC,
      out_shape=jax.ShapeDtypeStruct((NP, 128), f32),
  )(deg, xp)

  accA = _prop_half_edges(xt, srcA, dstA, z128).reshape(2, NP, 128)

  h1t = pl.pallas_call(
      _k2_body, grid=(GRID,),
      in_specs=[_DEG_SPEC, _ACC_SPEC, _ROW_SPEC,
                _full((128, 256)), _full((1, 256))],
      out_specs=_ACC_SPEC,
      out_shape=jax.ShapeDtypeStruct((2, NP, 128), f32),
  )(deg, accA, xt, W1, b1.reshape(1, 256))

  acc2 = _prop_half_cols(
      h1t.reshape(2 * NP, 128), srcB, dstB, z128).reshape(2, NP, 128)

  gt = pl.pallas_call(
      _k3_body, grid=(GRID,),
      in_specs=[_DEG_SPEC, _ACC_SPEC, _ACC_SPEC,
                _full((256, 256)), _full((1, 256)), _full((256, 128))],
      out_specs=_ROW_SPEC,
      out_shape=jax.ShapeDtypeStruct((NP, 128), f32),
  )(deg, acc2, h1t, W2, b2.reshape(1, 256), W3)

  acc3 = _prop_half_edges(gt, srcA, dstA, z128).reshape(2, NP, 128)

  out = pl.pallas_call(
      _k4_body, grid=(GRID,),
      in_specs=[_DEG_SPEC, _ACC_SPEC, _ROW_SPEC, _full((1, 128))],
      out_specs=_ROW_SPEC,
      out_shape=jax.ShapeDtypeStruct((NP, 128), f32),
  )(deg, acc3, gt, b3.reshape(1, 128))

  return out[:N]
